# baseline (speedup 1.0000x reference)
import functools

import jax
import jax.numpy as jnp
from jax import lax
from jax.experimental import pallas as pl
from jax.experimental.pallas import tpu as pltpu
from jax.experimental.pallas import tpu_sc as plsc

_B, _N = 32, 512
_TAU = 0.1
_C, _H, _W = 3, 224, 224
_IMG_D = _C * _H * _W
_MSK_D = _H * _W
_NCI = 8
_CWI = _IMG_D // _NCI
_NCM = 8
_CWM = _MSK_D // _NCM
_NC, _NS, _L = 2, 16, 16


def _tc_head(qe_ref, ge_ref, w1_ref, b1_ref, w2_ref, b2_ref, idx_ref, w_ref):
    h = jnp.dot(qe_ref[...], w1_ref[...], preferred_element_type=jnp.float32)
    h = jnp.maximum(h + b1_ref[...], 0.0)
    q = jnp.dot(h, w2_ref[...], preferred_element_type=jnp.float32) + b2_ref[...]
    qn = q / jnp.maximum(jnp.sqrt(jnp.sum(q * q, axis=1, keepdims=True)), 1e-8)
    g = ge_ref[...]
    gn = g / jnp.maximum(jnp.sqrt(jnp.sum(g * g, axis=1, keepdims=True)), 1e-8)
    sim = lax.dot_general(qn, gn, (((1,), (1,)), ((), ())),
                          preferred_element_type=jnp.float32)
    iota = lax.broadcasted_iota(jnp.int32, sim.shape, 1)
    m1 = jnp.max(sim, axis=1, keepdims=True)
    i1 = jnp.min(jnp.where(sim == m1, iota, _N), axis=1, keepdims=True)
    sim2 = jnp.where(iota == i1, -jnp.inf, sim)
    m2 = jnp.max(sim2, axis=1, keepdims=True)
    i2 = jnp.min(jnp.where(sim2 == m2, iota, _N), axis=1, keepdims=True)
    e = jnp.exp((m2 - m1) / _TAU)
    w0 = 1.0 / (1.0 + e)
    idx_ref[...] = jnp.concatenate([i1, i2], axis=1)
    w_ref[...] = jnp.concatenate([w0, 1.0 - w0], axis=1)


_sc_mesh = plsc.VectorSubcoreMesh(core_axis_name="c", subcore_axis_name="s")


@functools.partial(
    pl.kernel,
    out_type=(
        jax.ShapeDtypeStruct((_B * _NCI, _CWI), jnp.float32),
        jax.ShapeDtypeStruct((_B * _NCM, _CWM), jnp.float32),
    ),
    mesh=_sc_mesh,
    scratch_types=[
        pltpu.VMEM((_NCI, 2), jnp.int32),
        pltpu.VMEM((_NCM, 2), jnp.int32),
        pltpu.VMEM((2, _L), jnp.float32),
        pltpu.VMEM((2, _CWI), jnp.float32),
        pltpu.VMEM((_CWI,), jnp.float32),
        pltpu.VMEM((2, _CWM), jnp.float32),
        pltpu.VMEM((_CWM,), jnp.float32),
    ],
)
def _sc_combine(img_hbm, msk_hbm, idxi_hbm, idxm_hbm, w_hbm,
                oimg_hbm, omsk_hbm,
                idxi_v, idxm_v, w_v, ibuf_i, obuf_i, ibuf_m, obuf_m):
    b = lax.axis_index("s") * _NC + lax.axis_index("c")
    pltpu.sync_copy(idxi_hbm.at[b], idxi_v)
    pltpu.sync_copy(idxm_hbm.at[b], idxm_v)
    pltpu.sync_copy(w_hbm.at[b], w_v)
    w0 = w_v[0]
    w1 = w_v[1]

    @pl.loop(0, _NCI)
    def _img(c):
        pltpu.sync_copy(img_hbm.at[idxi_v.at[c]], ibuf_i)

        @pl.loop(0, _CWI, step=_L)
        def _(j):
            sl = pl.ds(j, _L)
            obuf_i[sl] = w0 * ibuf_i[0, sl] + w1 * ibuf_i[1, sl]

        pltpu.sync_copy(obuf_i, oimg_hbm.at[b * _NCI + c])

    @pl.loop(0, _NCM)
    def _msk(c):
        pltpu.sync_copy(msk_hbm.at[idxm_v.at[c]], ibuf_m)

        @pl.loop(0, _CWM, step=_L)
        def _(j):
            sl = pl.ds(j, _L)
            obuf_m[sl] = w0 * ibuf_m[0, sl] + w1 * ibuf_m[1, sl]

        pltpu.sync_copy(obuf_m, omsk_hbm.at[b * _NCM + c])


def kernel(query_encoding, gallery_embeddings, gallery_images, gallery_masks,
           W1, b1, W2, b2):
    idx, w = pl.pallas_call(
        _tc_head,
        out_shape=(
            jax.ShapeDtypeStruct((_B, 2), jnp.int32),
            jax.ShapeDtypeStruct((_B, 2), jnp.float32),
        ),
    )(query_encoding, gallery_embeddings, W1, b1.reshape(1, -1),
      W2, b2.reshape(1, -1))

    ii = idx[:, None, :] * _NCI + jnp.arange(_NCI, dtype=jnp.int32)[None, :, None]
    im = idx[:, None, :] * _NCM + jnp.arange(_NCM, dtype=jnp.int32)[None, :, None]
    wb = jnp.broadcast_to(w[:, :, None], (_B, 2, _L))

    oi, om = _sc_combine(
        gallery_images.reshape(_N * _NCI, _CWI),
        gallery_masks.reshape(_N * _NCM, _CWM),
        ii, im, wb)
    return oi.reshape(_B, _C, _H, _W), om.reshape(_B, _H, _W)


# speedup vs baseline: 1.1103x; 1.1103x over previous
import functools

import jax
import jax.numpy as jnp
from jax import lax
from jax.experimental import pallas as pl
from jax.experimental.pallas import tpu as pltpu
from jax.experimental.pallas import tpu_sc as plsc

_B, _N = 32, 512
_TAU = 0.1
_C, _H, _W = 3, 224, 224
_IMG_D = _C * _H * _W
_MSK_D = _H * _W
_NCI = 12
_CWI = _IMG_D // _NCI
_NCM = 8
_CWM = _MSK_D // _NCM
_NC, _NS, _L = 2, 16, 16
_UNROLL = 8


def _tc_head(qe_ref, ge_ref, w1_ref, b1_ref, w2_ref, b2_ref, idx_ref, w_ref):
    h = jnp.dot(qe_ref[...], w1_ref[...], preferred_element_type=jnp.float32)
    h = jnp.maximum(h + b1_ref[...], 0.0)
    q = jnp.dot(h, w2_ref[...], preferred_element_type=jnp.float32) + b2_ref[...]
    qn = q / jnp.maximum(jnp.sqrt(jnp.sum(q * q, axis=1, keepdims=True)), 1e-8)
    g = ge_ref[...]
    gn = g / jnp.maximum(jnp.sqrt(jnp.sum(g * g, axis=1, keepdims=True)), 1e-8)
    sim = lax.dot_general(qn, gn, (((1,), (1,)), ((), ())),
                          preferred_element_type=jnp.float32)
    iota = lax.broadcasted_iota(jnp.int32, sim.shape, 1)
    m1 = jnp.max(sim, axis=1, keepdims=True)
    i1 = jnp.min(jnp.where(sim == m1, iota, _N), axis=1, keepdims=True)
    sim2 = jnp.where(iota == i1, -jnp.inf, sim)
    m2 = jnp.max(sim2, axis=1, keepdims=True)
    i2 = jnp.min(jnp.where(sim2 == m2, iota, _N), axis=1, keepdims=True)
    e = jnp.exp((m2 - m1) / _TAU)
    w0 = 1.0 / (1.0 + e)
    idx_ref[...] = jnp.concatenate([i1, i2], axis=1)
    w_ref[...] = jnp.concatenate([w0, 1.0 - w0], axis=1)


_sc_mesh = plsc.VectorSubcoreMesh(core_axis_name="c", subcore_axis_name="s")


def _phase(tbl_hbm, out_hbm, idx_v, nch, cw, ibuf, obuf, out_base, w0, w1,
           gsems, osems):
    def gcopy(c):
        return pltpu.make_async_copy(
            tbl_hbm.at[idx_v.at[c]], ibuf.at[c % 2], gsems[c % 2])

    def ocopy(c):
        return pltpu.make_async_copy(
            obuf.at[c % 2], out_hbm.at[out_base + c], osems[c % 2])

    gcopy(0).start()
    for c in range(nch):
        p = c % 2
        if c + 1 < nch:
            gcopy(c + 1).start()
        gcopy(c).wait()
        if c >= 2:
            ocopy(c - 2).wait()
        src = ibuf.at[p]
        dst = obuf.at[p]

        @pl.loop(0, cw, step=_L * _UNROLL)
        def _(j):
            for u in range(_UNROLL):
                sl = pl.ds(j + u * _L, _L)
                dst[sl] = w0 * src[0, sl] + w1 * src[1, sl]

        ocopy(c).start()
    if nch >= 2:
        ocopy(nch - 2).wait()
    ocopy(nch - 1).wait()


@functools.partial(
    pl.kernel,
    out_type=(
        jax.ShapeDtypeStruct((_B * _NCI, _CWI), jnp.float32),
        jax.ShapeDtypeStruct((_B * _NCM, _CWM), jnp.float32),
    ),
    mesh=_sc_mesh,
    scratch_types=[
        pltpu.VMEM((_NCI, 2), jnp.int32),
        pltpu.VMEM((_NCM, 2), jnp.int32),
        pltpu.VMEM((2, _L), jnp.float32),
        pltpu.VMEM((2, 2, _CWI), jnp.float32),
        pltpu.VMEM((2, _CWI), jnp.float32),
        pltpu.VMEM((2, 2, _CWM), jnp.float32),
        pltpu.VMEM((2, _CWM), jnp.float32),
        pltpu.SemaphoreType.DMA,
        pltpu.SemaphoreType.DMA,
        pltpu.SemaphoreType.DMA,
        pltpu.SemaphoreType.DMA,
        pltpu.SemaphoreType.DMA,
        pltpu.SemaphoreType.DMA,
        pltpu.SemaphoreType.DMA,
        pltpu.SemaphoreType.DMA,
    ],
)
def _sc_combine(img_hbm, msk_hbm, idxi_hbm, idxm_hbm, w_hbm,
                oimg_hbm, omsk_hbm,
                idxi_v, idxm_v, w_v, ibuf_i, obuf_i, ibuf_m, obuf_m,
                gi0, gi1, oi0, oi1, gm0, gm1, om0, om1):
    b = lax.axis_index("s") * _NC + lax.axis_index("c")
    pltpu.sync_copy(idxi_hbm.at[b], idxi_v)
    pltpu.sync_copy(idxm_hbm.at[b], idxm_v)
    pltpu.sync_copy(w_hbm.at[b], w_v)
    w0 = w_v[0]
    w1 = w_v[1]
    _phase(img_hbm, oimg_hbm, idxi_v, _NCI, _CWI, ibuf_i, obuf_i,
           b * _NCI, w0, w1, (gi0, gi1), (oi0, oi1))
    _phase(msk_hbm, omsk_hbm, idxm_v, _NCM, _CWM, ibuf_m, obuf_m,
           b * _NCM, w0, w1, (gm0, gm1), (om0, om1))


def kernel(query_encoding, gallery_embeddings, gallery_images, gallery_masks,
           W1, b1, W2, b2):
    idx, w = pl.pallas_call(
        _tc_head,
        out_shape=(
            jax.ShapeDtypeStruct((_B, 2), jnp.int32),
            jax.ShapeDtypeStruct((_B, 2), jnp.float32),
        ),
    )(query_encoding, gallery_embeddings, W1, b1.reshape(1, -1),
      W2, b2.reshape(1, -1))

    ii = idx[:, None, :] * _NCI + jnp.arange(_NCI, dtype=jnp.int32)[None, :, None]
    im = idx[:, None, :] * _NCM + jnp.arange(_NCM, dtype=jnp.int32)[None, :, None]
    wb = jnp.broadcast_to(w[:, :, None], (_B, 2, _L))

    oi, om = _sc_combine(
        gallery_images.reshape(_N * _NCI, _CWI),
        gallery_masks.reshape(_N * _NCM, _CWM),
        ii, im, wb)
    return oi.reshape(_B, _C, _H, _W), om.reshape(_B, _H, _W)


# speedup vs baseline: 1.9116x; 1.7217x over previous
import dataclasses
import functools

import jax
import jax.numpy as jnp
from jax import lax
from jax.experimental import pallas as pl
from jax.experimental.pallas import tpu as pltpu
from jax.experimental.pallas import tpu_sc as plsc

_B, _N = 32, 512
_TAU = 0.1
_C, _H, _W = 3, 224, 224
_HB = 56
_NHB = _H // _HB
_NC, _NS, _L = 2, 16, 16


def _tc_head(qe_ref, ge_ref, w1_ref, b1_ref, w2_ref, b2_ref, idx_ref, w_ref):
    h = jnp.dot(qe_ref[...], w1_ref[...], preferred_element_type=jnp.float32)
    h = jnp.maximum(h + b1_ref[...], 0.0)
    q = jnp.dot(h, w2_ref[...], preferred_element_type=jnp.float32) + b2_ref[...]
    qn = q / jnp.maximum(jnp.sqrt(jnp.sum(q * q, axis=1, keepdims=True)), 1e-8)
    g = ge_ref[...]
    gn = g / jnp.maximum(jnp.sqrt(jnp.sum(g * g, axis=1, keepdims=True)), 1e-8)
    sim = lax.dot_general(qn, gn, (((1,), (1,)), ((), ())),
                          preferred_element_type=jnp.float32)
    iota = lax.broadcasted_iota(jnp.int32, sim.shape, 1)
    m1 = jnp.max(sim, axis=1, keepdims=True)
    i1 = jnp.min(jnp.where(sim == m1, iota, _N), axis=1, keepdims=True)
    sim2 = jnp.where(iota == i1, -jnp.inf, sim)
    m2 = jnp.max(sim2, axis=1, keepdims=True)
    i2 = jnp.min(jnp.where(sim2 == m2, iota, _N), axis=1, keepdims=True)
    e = jnp.exp((m2 - m1) / _TAU)
    w0 = 1.0 / (1.0 + e)
    idx_ref[...] = jnp.concatenate([i1, i2], axis=1)
    w_ref[...] = jnp.concatenate([w0, 1.0 - w0], axis=1)


_sc_mesh = plsc.VectorSubcoreMesh(core_axis_name="c", subcore_axis_name="s")

_sc_cp = pltpu.CompilerParams()
if "needs_layout_passes" in pltpu.CompilerParams.__dataclass_fields__:
    _sc_cp = dataclasses.replace(_sc_cp, needs_layout_passes=False)


@functools.partial(
    pl.kernel,
    compiler_params=_sc_cp,
    out_type=(
        jax.ShapeDtypeStruct((_B, _C, _H, _W), jnp.float32),
        jax.ShapeDtypeStruct((_B, _H, _W), jnp.float32),
    ),
    mesh=_sc_mesh,
    scratch_types=[
        pltpu.VMEM((2, _L), jnp.int32),
        pltpu.VMEM((2, _L), jnp.float32),
        pltpu.VMEM((2, 2, _HB, _W), jnp.float32),
        pltpu.VMEM((2, _HB, _W), jnp.float32),
        pltpu.SemaphoreType.DMA,
        pltpu.SemaphoreType.DMA,
        pltpu.SemaphoreType.DMA,
        pltpu.SemaphoreType.DMA,
    ],
)
def _sc_combine(img_hbm, msk_hbm, idx_hbm, w_hbm,
                oimg_hbm, omsk_hbm,
                idx_v, w_v, ibuf, obuf, g0, g1, o0, o1):
    b = lax.axis_index("s") * _NC + lax.axis_index("c")
    pltpu.sync_copy(idx_hbm.at[b], idx_v)
    pltpu.sync_copy(w_hbm.at[b], w_v)
    i0 = jnp.max(idx_v[0])
    i1 = jnp.max(idx_v[1])
    w0 = w_v[0]
    w1 = w_v[1]
    gsems = (g0, g1)
    osems = (o0, o1)

    tasks = []
    for ch in range(_C):
        for hb in range(_NHB):
            hsl = pl.ds(hb * _HB, _HB)
            tasks.append((
                lambda r, ch=ch, hsl=hsl: img_hbm.at[r, ch, hsl, :],
                oimg_hbm.at[b, ch, hsl, :],
            ))
    for hb in range(_NHB):
        hsl = pl.ds(hb * _HB, _HB)
        tasks.append((
            lambda r, hsl=hsl: msk_hbm.at[r, hsl, :],
            omsk_hbm.at[b, hsl, :],
        ))
    nt = len(tasks)

    def gstart(t):
        src, _ = tasks[t]
        p = t % 2
        pltpu.make_async_copy(src(i0), ibuf.at[p, 0], gsems[p]).start()
        pltpu.make_async_copy(src(i1), ibuf.at[p, 1], gsems[p]).start()

    def gwait(t):
        src, _ = tasks[t]
        p = t % 2
        pltpu.make_async_copy(src(i0), ibuf.at[p, 0], gsems[p]).wait()
        pltpu.make_async_copy(src(i1), ibuf.at[p, 1], gsems[p]).wait()

    def ocopy(t):
        _, dst = tasks[t]
        p = t % 2
        return pltpu.make_async_copy(obuf.at[p], dst, osems[p])

    gstart(0)
    for t in range(nt):
        p = t % 2
        if t + 1 < nt:
            gstart(t + 1)
        gwait(t)
        if t >= 2:
            ocopy(t - 2).wait()
        src = ibuf.at[p]
        dst = obuf.at[p]

        @pl.loop(0, _HB)
        def _(r):
            for u in range(_W // _L):
                sl = pl.ds(u * _L, _L)
                dst[r, sl] = w0 * src[0, r, sl] + w1 * src[1, r, sl]

        ocopy(t).start()
    ocopy(nt - 2).wait()
    ocopy(nt - 1).wait()


def kernel(query_encoding, gallery_embeddings, gallery_images, gallery_masks,
           W1, b1, W2, b2):
    idx, w = pl.pallas_call(
        _tc_head,
        out_shape=(
            jax.ShapeDtypeStruct((_B, 2), jnp.int32),
            jax.ShapeDtypeStruct((_B, 2), jnp.float32),
        ),
    )(query_encoding, gallery_embeddings, W1, b1.reshape(1, -1),
      W2, b2.reshape(1, -1))

    wb = jnp.broadcast_to(w[:, :, None], (_B, 2, _L))
    ib = jnp.broadcast_to(idx[:, :, None], (_B, 2, _L))
    return _sc_combine(gallery_images, gallery_masks, ib, wb)


# speedup vs baseline: 6.8014x; 3.5579x over previous
import jax
import jax.numpy as jnp
from jax import lax
from jax.experimental import pallas as pl

_B, _N = 32, 512
_TAU = 0.1
_C, _H, _W = 3, 224, 224
_HB = 8


def _tc_head(qe_ref, ge_ref, w1_ref, b1_ref, w2_ref, b2_ref, tw_ref):
    h = jnp.dot(qe_ref[...], w1_ref[...], preferred_element_type=jnp.float32)
    h = jnp.maximum(h + b1_ref[...], 0.0)
    q = jnp.dot(h, w2_ref[...], preferred_element_type=jnp.float32) + b2_ref[...]
    qn = q / jnp.maximum(jnp.sqrt(jnp.sum(q * q, axis=1, keepdims=True)), 1e-8)
    g = ge_ref[...]
    gn = g / jnp.maximum(jnp.sqrt(jnp.sum(g * g, axis=1, keepdims=True)), 1e-8)
    sim = lax.dot_general(qn, gn, (((1,), (1,)), ((), ())),
                          preferred_element_type=jnp.float32)
    iota = lax.broadcasted_iota(jnp.int32, sim.shape, 1)
    m1 = jnp.max(sim, axis=1, keepdims=True)
    i1 = jnp.min(jnp.where(sim == m1, iota, _N), axis=1, keepdims=True)
    sim2 = jnp.where(iota == i1, -jnp.inf, sim)
    m2 = jnp.max(sim2, axis=1, keepdims=True)
    i2 = jnp.min(jnp.where(sim2 == m2, iota, _N), axis=1, keepdims=True)
    e = jnp.exp((m2 - m1) / _TAU)
    w0 = 1.0 / (1.0 + e)
    tw_ref[...] = jnp.where(iota == i1, w0, 0.0) + jnp.where(iota == i2,
                                                             1.0 - w0, 0.0)


def _blend_both(tw_ref, img_ref, msk_ref, oimg_ref, omsk_ref):
    tw = tw_ref[...]
    ch = []
    for c in range(_C):
        hs = [lax.dot_general(tw, img_ref[c, h], (((1,), (1,)), ((), ())),
                              preferred_element_type=jnp.float32)
              for h in range(_HB)]
        ch.append(jnp.stack(hs, axis=1))
    oimg_ref[...] = jnp.stack(ch, axis=1)
    ms = [lax.dot_general(tw, msk_ref[h], (((1,), (1,)), ((), ())),
                          preferred_element_type=jnp.float32)
          for h in range(_HB)]
    omsk_ref[...] = jnp.stack(ms, axis=1)


def kernel(query_encoding, gallery_embeddings, gallery_images, gallery_masks,
           W1, b1, W2, b2):
    tw = pl.pallas_call(
        _tc_head,
        out_shape=jax.ShapeDtypeStruct((_B, _N), jnp.float32),
    )(query_encoding, gallery_embeddings, W1, b1.reshape(1, -1),
      W2, b2.reshape(1, -1))

    img_t = gallery_images.transpose(1, 2, 3, 0)
    msk_t = gallery_masks.transpose(1, 2, 0)

    oimg, omsk = pl.pallas_call(
        _blend_both,
        grid=(_H // _HB,),
        in_specs=[
            pl.BlockSpec((_B, _N), lambda hb: (0, 0)),
            pl.BlockSpec((_C, _HB, _W, _N), lambda hb: (0, hb, 0, 0)),
            pl.BlockSpec((_HB, _W, _N), lambda hb: (hb, 0, 0)),
        ],
        out_specs=(
            pl.BlockSpec((_B, _C, _HB, _W), lambda hb: (0, 0, hb, 0)),
            pl.BlockSpec((_B, _HB, _W), lambda hb: (0, hb, 0)),
        ),
        out_shape=(
            jax.ShapeDtypeStruct((_B, _C, _H, _W), jnp.float32),
            jax.ShapeDtypeStruct((_B, _H, _W), jnp.float32),
        ),
    )(tw, img_t, msk_t)

    return oimg, omsk
